# Initial kernel scaffold; baseline (speedup 1.0000x reference)
import jax
import jax.numpy as jnp
from jax.experimental import pallas as pl


def kernel(nodes, edges, senders, receivers, ew1, eb1, ew2, eb2, ew3, eb3, nw1, nb1, nw2, nb2, gw1, gb1, gw2, gb2, gw3, gb3):
    raise NotImplementedError("write your pallas kernel here")


# speedup vs baseline: 3.2261x; 3.2261x over previous
import functools
import jax
import jax.numpy as jnp
from jax import lax
from jax.experimental import pallas as pl
from jax.experimental.pallas import tpu as pltpu
from jax.experimental.pallas import tpu_sc as plsc

N = 100000
E = 1600000
NC, NS = 2, 16
NW = NC * NS
EPW = E // NW
SUB = 125
K = 16
CHUNK = SUB * K
NCHUNK = EPW // CHUNK
ROWS2D = E // SUB
NPAD = 100352
NPS = NPAD // NS
SK = 10
SCHUNK = SUB * SK
SNCHUNK = EPW // SCHUNK

_SELU_ALPHA = 1.6732632423543772
_SELU_SCALE = 1.0507009873554805


def _selu(x):
    return _SELU_SCALE * jnp.where(x > 0, x, _SELU_ALPHA * (jnp.exp(x) - 1.0))


def _gather_body(nodes16, sidx, ridx, sfeat, rfeat, idx_v, rows_v, sem):
    cid = lax.axis_index("c")
    sid = lax.axis_index("s")
    wid = cid * NS + sid
    r0 = wid * (EPW // SUB)

    def chunk(ci, carry):
        off2d = r0 + ci * K
        off1d = wid * EPW + ci * CHUNK
        for idx_hbm, out_hbm in ((sidx, sfeat), (ridx, rfeat)):
            pltpu.sync_copy(idx_hbm.at[pl.ds(off2d, K)], idx_v)
            cps = []
            for j in range(K):
                cps.append(pltpu.async_copy(
                    nodes16.at[idx_v.at[j]],
                    rows_v.at[pl.ds(j * SUB, SUB)], sem))
            for cp in cps:
                cp.wait()
            pltpu.sync_copy(rows_v, out_hbm.at[pl.ds(off1d, CHUNK)])
        return carry

    lax.fori_loop(0, NCHUNK, chunk, 0)


def _gather(nodes16, sidx, ridx):
    mesh = plsc.VectorSubcoreMesh(core_axis_name="c", subcore_axis_name="s",
                                  num_cores=NC, num_subcores=NS)
    fn = pl.kernel(
        _gather_body,
        out_type=(jax.ShapeDtypeStruct((E, 16), jnp.float32),
                  jax.ShapeDtypeStruct((E, 16), jnp.float32)),
        mesh=mesh,
        compiler_params=pltpu.CompilerParams(use_tc_tiling_on_sc=False),
        scratch_types=[
            pltpu.VMEM((K, SUB), jnp.int32),
            pltpu.VMEM((CHUNK, 16), jnp.float32),
            pltpu.SemaphoreType.DMA,
        ],
    )
    return fn(nodes16, sidx, ridx)


def _scatter_body(e16, ridx, zeros16, agg0, agg1, acc, idx_v, ebuf, sem):
    cid = lax.axis_index("c")
    sid = lax.axis_index("s")
    wid = cid * NS + sid
    r0 = wid * (EPW // SUB)

    pltpu.sync_copy(zeros16, acc.at[pl.ds(sid * NPS, NPS)])
    plsc.subcore_barrier()

    def chunk(ci, carry):
        off2d = r0 + ci * SK
        off1d = wid * EPW + ci * SCHUNK
        pltpu.sync_copy(ridx.at[pl.ds(off2d, SK)], idx_v)
        pltpu.sync_copy(e16.at[pl.ds(off1d, SCHUNK)], ebuf)
        cps = []
        for j in range(SK):
            cps.append(pltpu.async_copy(
                ebuf.at[pl.ds(j * SUB, SUB)],
                acc.at[idx_v.at[j]], sem, add=True))
        for cp in cps:
            cp.wait()
        return carry

    lax.fori_loop(0, SNCHUNK, chunk, 0)
    plsc.subcore_barrier()

    @pl.when(cid == 0)
    def _():
        pltpu.sync_copy(acc.at[pl.ds(sid * NPS, NPS)],
                        agg0.at[pl.ds(sid * NPS, NPS)])

    @pl.when(cid == 1)
    def _():
        pltpu.sync_copy(acc.at[pl.ds(sid * NPS, NPS)],
                        agg1.at[pl.ds(sid * NPS, NPS)])


def _scatter(e16, ridx, zeros16):
    mesh = plsc.VectorSubcoreMesh(core_axis_name="c", subcore_axis_name="s",
                                  num_cores=NC, num_subcores=NS)
    fn = pl.kernel(
        _scatter_body,
        out_type=(jax.ShapeDtypeStruct((NPAD, 16), jnp.float32),
                  jax.ShapeDtypeStruct((NPAD, 16), jnp.float32)),
        mesh=mesh,
        compiler_params=pltpu.CompilerParams(use_tc_tiling_on_sc=False),
        scratch_types=[
            pltpu.VMEM_SHARED((NPAD, 16), jnp.float32),
            pltpu.VMEM((SK, SUB), jnp.int32),
            pltpu.VMEM((SCHUNK, 16), jnp.float32),
            pltpu.SemaphoreType.DMA,
        ],
    )
    return fn(e16, ridx, zeros16)


BE = 4000
NBE = E // BE


def _edge_mlp_body(rfeat, sfeat, edges8, wr, ws, we, b1, w2, b2, w3p, b3p,
                   e10, e16, esum):
    h = (rfeat[...] @ wr[...] + sfeat[...] @ ws[...]
         + edges8[...] @ we[...] + b1[...])
    h = _selu(h)
    h = _selu(h @ w2[...] + b2[...])
    out16 = h @ w3p[...] + b3p[...]
    e16[...] = out16
    e10[...] = out16[:, :10]
    esum[...] = jnp.sum(out16, axis=0, keepdims=True)[:, None, :]


def _edge_mlp(rfeat, sfeat, edges8, wr, ws, we, b1, w2, b2, w3p, b3p):
    full = lambda s: pl.BlockSpec(s, lambda i: (0,) * len(s))
    return pl.pallas_call(
        _edge_mlp_body,
        grid=(NBE,),
        in_specs=[
            pl.BlockSpec((BE, 16), lambda i: (i, 0)),
            pl.BlockSpec((BE, 16), lambda i: (i, 0)),
            pl.BlockSpec((BE, 8), lambda i: (i, 0)),
            full((16, 25)), full((16, 25)), full((8, 25)), full((1, 25)),
            full((25, 20)), full((1, 20)), full((20, 16)), full((1, 16)),
        ],
        out_specs=[
            pl.BlockSpec((BE, 10), lambda i: (i, 0)),
            pl.BlockSpec((BE, 16), lambda i: (i, 0)),
            pl.BlockSpec((1, 1, 16), lambda i: (i, 0, 0)),
        ],
        out_shape=[
            jax.ShapeDtypeStruct((E, 10), jnp.float32),
            jax.ShapeDtypeStruct((E, 16), jnp.float32),
            jax.ShapeDtypeStruct((NBE, 1, 16), jnp.float32),
        ],
    )(rfeat, sfeat, edges8, wr, ws, we, b1, w2, b2, w3p, b3p)


BN = 4000
NBN = N // BN


def _node_mlp_body(agg0, agg1, nodes16, w1a, w1b, b1, w2p, b2p, n10, nsum):
    agg = agg0[...] + agg1[...]
    h = _selu(agg @ w1a[...] + nodes16[...] @ w1b[...] + b1[...])
    out16 = h @ w2p[...] + b2p[...]
    n10[...] = out16[:, :10]
    nsum[...] = jnp.sum(out16, axis=0, keepdims=True)[:, None, :]


def _node_mlp(agg0, agg1, nodes16, w1a, w1b, b1, w2p, b2p):
    full = lambda s: pl.BlockSpec(s, lambda i: (0,) * len(s))
    return pl.pallas_call(
        _node_mlp_body,
        grid=(NBN,),
        in_specs=[
            pl.BlockSpec((BN, 16), lambda i: (i, 0)),
            pl.BlockSpec((BN, 16), lambda i: (i, 0)),
            pl.BlockSpec((BN, 16), lambda i: (i, 0)),
            full((16, 18)), full((16, 18)), full((1, 18)),
            full((18, 16)), full((1, 16)),
        ],
        out_specs=[
            pl.BlockSpec((BN, 10), lambda i: (i, 0)),
            pl.BlockSpec((1, 1, 16), lambda i: (i, 0, 0)),
        ],
        out_shape=[
            jax.ShapeDtypeStruct((N, 10), jnp.float32),
            jax.ShapeDtypeStruct((NBN, 1, 16), jnp.float32),
        ],
    )(agg0, agg1, nodes16, w1a, w1b, b1, w2p, b2p)


def _global_body(esums, nsums, gw1, gb1, gw2, gb2, gw3, gb3, gout):
    es = jnp.sum(esums[...], axis=(0, 1), keepdims=False)[None, :]
    ns = jnp.sum(nsums[...], axis=(0, 1), keepdims=False)[None, :]
    gin = jnp.concatenate([es[:, :10], ns[:, :10]], axis=1)
    h = _selu(gin @ gw1[...] + gb1[...])
    h = _selu(h @ gw2[...] + gb2[...])
    gout[...] = h @ gw3[...] + gb3[...]


def _global(esums, nsums, gw1, gb1, gw2, gb2, gw3, gb3):
    return pl.pallas_call(
        _global_body,
        out_shape=jax.ShapeDtypeStruct((1, 10), jnp.float32),
    )(esums, nsums, gw1, gb1, gw2, gb2, gw3, gb3)


def kernel(nodes, edges, senders, receivers,
           ew1, eb1, ew2, eb2, ew3, eb3,
           nw1, nb1, nw2, nb2,
           gw1, gb1, gw2, gb2, gw3, gb3):
    f32 = jnp.float32
    nodes16 = jnp.pad(nodes, ((0, 0), (0, 3)))
    edges8 = jnp.pad(edges, ((0, 0), (0, 3)))
    sidx = senders.reshape(ROWS2D, SUB)
    ridx = receivers.reshape(ROWS2D, SUB)

    sfeat, rfeat = _gather(nodes16, sidx, ridx)

    z3 = jnp.zeros((3, 25), f32)
    we = jnp.concatenate([ew1[0:5], jnp.zeros((3, 25), f32)], axis=0)
    wr = jnp.concatenate([ew1[5:18], z3], axis=0)
    ws = jnp.concatenate([ew1[18:31], z3], axis=0)
    w3p = jnp.pad(ew3, ((0, 0), (0, 6)))
    b3p = jnp.pad(eb3, (0, 6))[None, :]
    e10, e16, esums = _edge_mlp(rfeat, sfeat, edges8, wr, ws, we,
                                eb1[None, :], ew2, eb2[None, :], w3p, b3p)

    zeros16 = jnp.zeros((NPS, 16), f32)
    agg0, agg1 = _scatter(e16, ridx, zeros16)

    w1a = jnp.concatenate([nw1[0:10], jnp.zeros((6, 18), f32)], axis=0)
    w1b = jnp.concatenate([nw1[10:23], jnp.zeros((3, 18), f32)], axis=0)
    w2p = jnp.pad(nw2, ((0, 0), (0, 6)))
    b2p = jnp.pad(nb2, (0, 6))[None, :]
    n10, nsums = _node_mlp(agg0, agg1, nodes16, w1a, w1b,
                           nb1[None, :], w2p, b2p)

    g = _global(esums, nsums, gw1, gb1[None, :], gw2, gb2[None, :],
                gw3, gb3[None, :])
    return (n10, e10, g)
